# baseline (speedup 1.0000x reference)
import functools

import jax
import jax.numpy as jnp
from jax.experimental import pallas as pl
from jax.experimental.pallas import tpu as pltpu

BLOCK_N = 1024
NBUF = 8


def _matmul_bias_kernel(x_hbm, wt_ref, b_ref, o_ref, buf, sems):
    n = o_ref.shape[0]
    nblk = n // BLOCK_N

    def copy_in(blk, slot):
        return pltpu.make_async_copy(
            x_hbm.at[pl.ds(blk * BLOCK_N, BLOCK_N), :],
            buf.at[slot],
            sems.at[slot],
        )

    for j in range(NBUF):
        copy_in(j, j).start()

    def step(i, carry):
        slot = jax.lax.rem(i, NBUF)
        copy_in(i, slot).wait()
        o_ref[pl.ds(i * BLOCK_N, BLOCK_N), :] = (
            jnp.dot(buf[slot], wt_ref[...], preferred_element_type=jnp.float32)
            + b_ref[...]
        )

        @pl.when(i + NBUF < nblk)
        def _():
            copy_in(i + NBUF, slot).start()

        return carry

    jax.lax.fori_loop(0, nblk, step, 0)


@jax.jit
def kernel(x, W_last, b_last, W_dom, b_dom):
    xs = jnp.squeeze(x)
    n, k = xs.shape
    m = W_last.shape[0]
    wt = W_last.T
    b2 = b_last.reshape(1, m)
    return pl.pallas_call(
        _matmul_bias_kernel,
        in_specs=[
            pl.BlockSpec(memory_space=pltpu.MemorySpace.HBM),
            pl.BlockSpec((k, m), lambda: (0, 0)),
            pl.BlockSpec((1, m), lambda: (0, 0)),
        ],
        out_specs=pl.BlockSpec((n, m), lambda: (0, 0)),
        out_shape=jax.ShapeDtypeStruct((n, m), jnp.float32),
        scratch_shapes=[
            pltpu.VMEM((NBUF, BLOCK_N, k), jnp.float32),
            pltpu.SemaphoreType.DMA((NBUF,)),
        ],
    )(xs, wt, b2)
